# baseline (device time: 20251 ns/iter reference)
import functools

import jax
import jax.numpy as jnp
from jax import lax
from jax.experimental import pallas as pl
from jax.experimental.pallas import tpu as pltpu

M_PER = 2048
D = 2048
BLOCK = 256
N_BLOCKS = M_PER // BLOCK
EPS = 1e-5


def _body(x_ref, dy_ref, out_ref, acc_ref, peer_ref, send_sem, recv_sem):
    i = pl.program_id(0)

    @pl.when(i == 0)
    def _():
        acc_ref[...] = jnp.zeros_like(acc_ref)

    x = x_ref[...]
    dy = dy_ref[...]
    mu = jnp.mean(x, axis=-1, keepdims=True)
    var = jnp.mean((x - mu) * (x - mu), axis=-1, keepdims=True)
    rstd = lax.rsqrt(var + EPS)
    xhat = (x - mu) * rstd
    acc_ref[0:1, :] += jnp.sum(dy * xhat, axis=0, keepdims=True)
    acc_ref[1:2, :] += jnp.sum(dy, axis=0, keepdims=True)

    @pl.when(i == N_BLOCKS - 1)
    def _():
        my_x = lax.axis_index("x")
        my_y = lax.axis_index("y")
        my_z = lax.axis_index("z")
        peer = (1 - my_x, my_y, my_z)

        barrier_sem = pltpu.get_barrier_semaphore()
        pl.semaphore_signal(
            barrier_sem, inc=1,
            device_id=peer, device_id_type=pl.DeviceIdType.MESH,
        )
        pl.semaphore_wait(barrier_sem, 1)

        rdma = pltpu.make_async_remote_copy(
            src_ref=acc_ref,
            dst_ref=peer_ref,
            send_sem=send_sem,
            recv_sem=recv_sem,
            device_id=peer,
            device_id_type=pl.DeviceIdType.MESH,
        )
        rdma.start()
        rdma.wait()

        out_ref[...] = acc_ref[...] + peer_ref[...]


def kernel(x, dy, gamma):
    del gamma
    return pl.pallas_call(
        _body,
        grid=(N_BLOCKS,),
        in_specs=[
            pl.BlockSpec((BLOCK, D), lambda i: (i, 0)),
            pl.BlockSpec((BLOCK, D), lambda i: (i, 0)),
        ],
        out_specs=pl.BlockSpec((2, D), lambda i: (0, 0)),
        out_shape=jax.ShapeDtypeStruct((2, D), jnp.float32),
        scratch_shapes=[
            pltpu.VMEM((2, D), jnp.float32),
            pltpu.VMEM((2, D), jnp.float32),
            pltpu.SemaphoreType.DMA,
            pltpu.SemaphoreType.DMA,
        ],
        compiler_params=pltpu.CompilerParams(collective_id=0),
    )(x, dy)


# device time: 16873 ns/iter; 1.2002x vs baseline; 1.2002x over previous
import jax
import jax.numpy as jnp
from jax import lax
from jax.experimental import pallas as pl
from jax.experimental.pallas import tpu as pltpu

M_PER = 2048
D = 2048
N_DEV = 16
N_SPLIT = 8
ROWS = M_PER // N_SPLIT
EPS = 1e-5

_COORDS = [(xx, yy, zz) for xx in range(2) for yy in range(2) for zz in range(4)]


def _body(x_hbm, dy_hbm, out_ref, xb, dyb, inbox, load_sems,
          send_sems, recv_sems, credit_sems):
    my_x = lax.axis_index("x")
    my_y = lax.axis_index("y")
    my_z = lax.axis_index("z")
    my_id = my_x * 8 + my_y * 4 + my_z
    rid = my_y * 4 + my_z
    row0 = rid * ROWS

    cp_x = pltpu.make_async_copy(
        x_hbm.at[pl.ds(row0, ROWS), :], xb, load_sems.at[0])
    cp_dy = pltpu.make_async_copy(
        dy_hbm.at[pl.ds(row0, ROWS), :], dyb, load_sems.at[1])
    cp_x.start()
    cp_dy.start()

    barrier_sem = pltpu.get_barrier_semaphore()
    for pid in range(N_DEV):
        @pl.when(pid != my_id)
        def _():
            pl.semaphore_signal(
                barrier_sem, inc=1,
                device_id=_COORDS[pid], device_id_type=pl.DeviceIdType.MESH)
            pl.semaphore_signal(
                credit_sems.at[my_id], inc=1,
                device_id=_COORDS[pid], device_id_type=pl.DeviceIdType.MESH)
    pl.semaphore_wait(barrier_sem, N_DEV - 1)

    cp_x.wait()
    cp_dy.wait()
    x = xb[...]
    dy = dyb[...]
    mu = jnp.mean(x, axis=-1, keepdims=True)
    var = jnp.mean((x - mu) * (x - mu), axis=-1, keepdims=True)
    rstd = lax.rsqrt(var + EPS)
    xhat = (x - mu) * rstd
    dgamma = jnp.sum(dy * xhat, axis=0, keepdims=True)
    dbeta = jnp.sum(dy, axis=0, keepdims=True)
    inbox[pl.ds(my_id, 1)] = jnp.concatenate([dgamma, dbeta], axis=0)[None]

    for pid in range(N_DEV):
        @pl.when(pid != my_id)
        def _():
            pl.semaphore_wait(credit_sems.at[pid], 1)
            rdma = pltpu.make_async_remote_copy(
                src_ref=inbox.at[my_id],
                dst_ref=inbox.at[my_id],
                send_sem=send_sems.at[pid],
                recv_sem=recv_sems.at[my_id],
                device_id=_COORDS[pid],
                device_id_type=pl.DeviceIdType.MESH,
            )
            rdma.start()

    for pid in range(N_DEV):
        @pl.when(pid != my_id)
        def _():
            recv = pltpu.make_async_remote_copy(
                src_ref=inbox.at[pid],
                dst_ref=inbox.at[pid],
                send_sem=send_sems.at[pid],
                recv_sem=recv_sems.at[pid],
                device_id=_COORDS[pid],
                device_id_type=pl.DeviceIdType.MESH,
            )
            recv.wait_recv()
            send = pltpu.make_async_remote_copy(
                src_ref=inbox.at[my_id],
                dst_ref=inbox.at[my_id],
                send_sem=send_sems.at[pid],
                recv_sem=recv_sems.at[my_id],
                device_id=_COORDS[pid],
                device_id_type=pl.DeviceIdType.MESH,
            )
            send.wait_send()

    out_ref[...] = jnp.sum(inbox[...], axis=0)


def kernel(x, dy, gamma):
    del gamma
    return pl.pallas_call(
        _body,
        in_specs=[
            pl.BlockSpec(memory_space=pltpu.MemorySpace.HBM),
            pl.BlockSpec(memory_space=pltpu.MemorySpace.HBM),
        ],
        out_specs=pl.BlockSpec(memory_space=pltpu.VMEM),
        out_shape=jax.ShapeDtypeStruct((2, D), jnp.float32),
        scratch_shapes=[
            pltpu.VMEM((ROWS, D), jnp.float32),
            pltpu.VMEM((ROWS, D), jnp.float32),
            pltpu.VMEM((N_DEV, 2, D), jnp.float32),
            pltpu.SemaphoreType.DMA((2,)),
            pltpu.SemaphoreType.DMA((N_DEV,)),
            pltpu.SemaphoreType.DMA((N_DEV,)),
            pltpu.SemaphoreType.REGULAR((N_DEV,)),
        ],
        compiler_params=pltpu.CompilerParams(collective_id=0),
    )(x, dy)


# device time: 13111 ns/iter; 1.5446x vs baseline; 1.2869x over previous
import jax
import jax.numpy as jnp
from jax import lax
from jax.experimental import pallas as pl
from jax.experimental.pallas import tpu as pltpu

M_PER = 2048
D = 2048
N_DEV = 16
N_SPLIT = 8
ROWS = M_PER // N_SPLIT
EPS = 1e-5

_COORDS = [(xx, yy, zz) for xx in range(2) for yy in range(2) for zz in range(4)]


def _body(x_vmem, dy_vmem, out_ref, inbox, send_sems, recv_sems, credit_sems):
    my_x = lax.axis_index("x")
    my_y = lax.axis_index("y")
    my_z = lax.axis_index("z")
    my_id = my_x * 8 + my_y * 4 + my_z

    barrier_sem = pltpu.get_barrier_semaphore()
    pl.semaphore_signal(
        barrier_sem, inc=1,
        device_id=(1 - my_x, my_y, my_z),
        device_id_type=pl.DeviceIdType.MESH)
    pl.semaphore_wait(barrier_sem, 1)

    for pid in range(N_DEV):
        @pl.when(pid != my_id)
        def _():
            pl.semaphore_signal(
                credit_sems.at[my_id], inc=1,
                device_id=_COORDS[pid], device_id_type=pl.DeviceIdType.MESH)

    x = x_vmem[...]
    dy = dy_vmem[...]
    mu = jnp.mean(x, axis=-1, keepdims=True)
    var = jnp.mean((x - mu) * (x - mu), axis=-1, keepdims=True)
    rstd = lax.rsqrt(var + EPS)
    xhat = (x - mu) * rstd
    dgamma = jnp.sum(dy * xhat, axis=0, keepdims=True)
    dbeta = jnp.sum(dy, axis=0, keepdims=True)
    inbox[pl.ds(my_id, 1)] = jnp.concatenate([dgamma, dbeta], axis=0)[None]

    for pid in range(N_DEV):
        @pl.when(pid != my_id)
        def _():
            pl.semaphore_wait(credit_sems.at[pid], 1)
            rdma = pltpu.make_async_remote_copy(
                src_ref=inbox.at[my_id],
                dst_ref=inbox.at[my_id],
                send_sem=send_sems.at[pid],
                recv_sem=recv_sems.at[my_id],
                device_id=_COORDS[pid],
                device_id_type=pl.DeviceIdType.MESH,
            )
            rdma.start()

    for pid in range(N_DEV):
        @pl.when(pid != my_id)
        def _():
            recv = pltpu.make_async_remote_copy(
                src_ref=inbox.at[pid],
                dst_ref=inbox.at[pid],
                send_sem=send_sems.at[pid],
                recv_sem=recv_sems.at[pid],
                device_id=_COORDS[pid],
                device_id_type=pl.DeviceIdType.MESH,
            )
            recv.wait_recv()

    out_ref[...] = jnp.sum(inbox[...], axis=0)

    for pid in range(N_DEV):
        @pl.when(pid != my_id)
        def _():
            send = pltpu.make_async_remote_copy(
                src_ref=inbox.at[my_id],
                dst_ref=inbox.at[my_id],
                send_sem=send_sems.at[pid],
                recv_sem=recv_sems.at[my_id],
                device_id=_COORDS[pid],
                device_id_type=pl.DeviceIdType.MESH,
            )
            send.wait_send()


def kernel(x, dy, gamma):
    del gamma
    rid = lax.axis_index("y") * 4 + lax.axis_index("z")
    x_sl = lax.dynamic_slice(x, (rid * ROWS, 0), (ROWS, D))
    dy_sl = lax.dynamic_slice(dy, (rid * ROWS, 0), (ROWS, D))
    return pl.pallas_call(
        _body,
        in_specs=[
            pl.BlockSpec(memory_space=pltpu.MemorySpace.VMEM),
            pl.BlockSpec(memory_space=pltpu.MemorySpace.VMEM),
        ],
        out_specs=pl.BlockSpec(memory_space=pltpu.MemorySpace.VMEM),
        out_shape=jax.ShapeDtypeStruct((2, D), jnp.float32),
        scratch_shapes=[
            pltpu.VMEM((N_DEV, 2, D), jnp.float32),
            pltpu.SemaphoreType.DMA((N_DEV,)),
            pltpu.SemaphoreType.DMA((N_DEV,)),
            pltpu.SemaphoreType.REGULAR((N_DEV,)),
        ],
        compiler_params=pltpu.CompilerParams(collective_id=0),
    )(x_sl, dy_sl)
